# baseline (speedup 1.0000x reference)
import jax
import jax.numpy as jnp
from jax import lax
from jax.experimental import pallas as pl
from jax.experimental.pallas import tpu as pltpu
from jax.experimental.pallas import tpu_sc as plsc

NX, NY, NZ = 128, 128, 256
N = NX * NY
NNZ = 268435

NC = 2
NS = 16
CW = 32
NCHUNK = NZ // CW
CHUNKS_PER_SC = NCHUNK // NC
BATCH = 128
K = 3
PD = 2
R = 2
RB = R * BATCH
M = ((-(-NNZ // (NS * RB)) + K - 1) // K) * K
NNZ_PAD = NS * M * RB
ROWS_PER_TILE = N // NS


def _mm_body(x_ref, w_ref, o_ref):
    o_ref[...] = jnp.dot(x_ref[...], w_ref[...],
                         preferred_element_type=jnp.float32
                         ).astype(jnp.bfloat16)


def _matmul(squ, mat_z):
    BM = 1024
    return pl.pallas_call(
        _mm_body,
        grid=(N // BM,),
        in_specs=[
            pl.BlockSpec((BM, NZ), lambda i: (i, 0)),
            pl.BlockSpec((NZ, NZ), lambda i: (0, 0)),
        ],
        out_specs=pl.BlockSpec((BM, NZ), lambda i: (i, 0)),
        out_shape=jax.ShapeDtypeStruct((N, NZ), jnp.bfloat16),
    )(squ, mat_z)


def _sc_body(table, packed_h, vals_h, out,
             idx_v, cols_v, vals_v,
             bb0, bb1, bb2, fb0, fb1, fb2, acc_ref, tbl_s,
             gs0, gs1, gs2, ss0, ss1, ss2):
    c = lax.axis_index("c")
    s = lax.axis_index("s")
    bbuf = (bb0, bb1, bb2)
    fbuf = (fb0, fb1, fb2)
    gsem = (gs0, gs1, gs2)
    ssem = (ss0, ss1, ss2)

    pltpu.sync_copy(packed_h.at[s], idx_v)
    pltpu.sync_copy(vals_h.at[s], vals_v)

    maskv = jnp.full((16,), N - 1, jnp.int32)
    zvec = jnp.zeros((16,), jnp.float32)

    def _unpack_body(b, _):
        for j in range(RB // 16):
            w = idx_v[b, pl.ds(j * 16, 16)]
            cols_v[b, pl.ds(j * 16, 16)] = w & maskv
            idx_v[b, pl.ds(j * 16, 16)] = lax.shift_right_logical(
                w, jnp.full((16,), 14, jnp.int32))
        return 0

    lax.fori_loop(0, M, _unpack_body, 0)

    dnums = lax.GatherDimensionNumbers(
        offset_dims=(), collapsed_slice_dims=(0,), start_index_map=(0,))

    def _splat(vec, l):
        return lax.gather(
            vec, jnp.full((16, 1), l, jnp.int32), dnums, (1,),
            mode=lax.GatherScatterMode.PROMISE_IN_BOUNDS)

    for i in range(CHUNKS_PER_SC):
        chunk = CHUNKS_PER_SC * c + i
        pltpu.sync_copy(
            table.at[pl.ds(s * ROWS_PER_TILE, ROWS_PER_TILE),
                     pl.ds(chunk * CW, CW)],
            tbl_s.at[pl.ds(s * ROWS_PER_TILE, ROWS_PER_TILE)])

        def _zero_body(r, _):
            for k in range(CW // 16):
                fb0[r, pl.ds(k * 16, 16)] = zvec
            return 0

        lax.fori_loop(0, BATCH, _zero_body, 0)
        for z in range(ROWS_PER_TILE // BATCH):
            pltpu.sync_copy(
                fb0.at[pl.ds(0, BATCH)],
                acc_ref.at[pl.ds(s * ROWS_PER_TILE + z * BATCH, BATCH)])
        plsc.subcore_barrier()

        for k in range(PD):
            pltpu.async_copy(tbl_s.at[idx_v.at[k]], bbuf[k], gsem[k])

        def _scale(bb, fb, m):
            def _rows(g, _):
                raws = vals_v[m, pl.ds(g * 32, 32)]
                vlo, vhi = plsc.unpack(
                    raws, format=plsc.PackFormat.INTERLEAVED)
                for j2 in range(16):
                    for vv, off in ((vlo, 0), (vhi, 1)):
                        vsp = _splat(vv, j2)
                        r = g * 32 + 2 * j2 + off
                        raw = bb[r, pl.ds(0, CW)]
                        lo, hi = plsc.unpack(
                            raw, format=plsc.PackFormat.INTERLEAVED)
                        fb[r, pl.ds(0, 16)] = lo * vsp
                        fb[r, pl.ds(16, 16)] = hi * vsp
                return 0
            lax.fori_loop(0, RB // 32, _rows, 0)

        def _ring_body(g, _):
            for k in range(K):
                m = g * K + k
                pltpu.make_async_copy(tbl_s.at[idx_v.at[m]],
                                      bbuf[k], gsem[k]).wait()

                @pl.when(m - K >= 0)
                def _():
                    pltpu.make_async_copy(
                        fbuf[k], acc_ref.at[cols_v.at[m - K]],
                        ssem[k]).wait()
                _scale(bbuf[k], fbuf[k], m)
                pltpu.async_copy(
                    fbuf[k], acc_ref.at[cols_v.at[m]], ssem[k], add=True)

                mp = m + PD
                kp = (k + PD) % K

                @pl.when(mp < M)
                def _():
                    pltpu.async_copy(tbl_s.at[idx_v.at[mp]], bbuf[kp],
                                     gsem[kp])
            return 0

        lax.fori_loop(0, M // K, _ring_body, 0)
        for k in range(K):
            m = M - K + k
            pltpu.make_async_copy(
                fbuf[m % K], acc_ref.at[cols_v.at[m]],
                ssem[m % K]).wait()
        plsc.subcore_barrier()

        pltpu.sync_copy(
            acc_ref.at[pl.ds(s * ROWS_PER_TILE, ROWS_PER_TILE)],
            out.at[pl.ds(s * ROWS_PER_TILE, ROWS_PER_TILE),
                   pl.ds(chunk * CW, CW)])
        plsc.subcore_barrier()


def _sc_scatter(table, packed_r, vals_r):
    mesh = plsc.VectorSubcoreMesh(core_axis_name="c", subcore_axis_name="s")
    f = pl.kernel(
        _sc_body,
        mesh=mesh,
        compiler_params=pltpu.CompilerParams(use_tc_tiling_on_sc=False,
                                             needs_layout_passes=False),
        out_type=jax.ShapeDtypeStruct((N, NZ), jnp.float32),
        scratch_types=[
            pltpu.VMEM((M, RB), jnp.int32),
            pltpu.VMEM((M, RB), jnp.int32),
            pltpu.VMEM((M, RB), jnp.bfloat16),
            pltpu.VMEM((RB, CW), jnp.bfloat16),
            pltpu.VMEM((RB, CW), jnp.bfloat16),
            pltpu.VMEM((RB, CW), jnp.bfloat16),
            pltpu.VMEM((RB, CW), jnp.float32),
            pltpu.VMEM((RB, CW), jnp.float32),
            pltpu.VMEM((RB, CW), jnp.float32),
            pltpu.VMEM_SHARED((N, CW), jnp.float32),
            pltpu.VMEM_SHARED((N, CW), jnp.bfloat16),
            pltpu.SemaphoreType.DMA,
            pltpu.SemaphoreType.DMA,
            pltpu.SemaphoreType.DMA,
            pltpu.SemaphoreType.DMA,
            pltpu.SemaphoreType.DMA,
            pltpu.SemaphoreType.DMA,
        ],
    )
    return f(table, packed_r, vals_r)


def _mat_z_perm():
    perm = []
    for cb in range(0, NZ, CW):
        for j in range(CW // 2):
            perm.append(cb + j)
            perm.append(cb + CW // 2 + j)
    return perm


_PERM = _mat_z_perm()


def kernel(bproj, mat_xy_indices, mat_xy_values, mat_z):
    squ = bproj.reshape(N, NZ)
    result1 = _matmul(squ, mat_z[:, jnp.array(_PERM, dtype=jnp.int32)])
    table = result1

    rows = mat_xy_indices[:, 0]
    cols = mat_xy_indices[:, 1]
    packed = rows * N + cols
    pad = NNZ_PAD - NNZ
    pad_idx = (jnp.arange(pad, dtype=jnp.int32) * 37) % N
    packed_p = jnp.concatenate(
        [packed, pad_idx * N + pad_idx]).reshape(NS, M, RB)
    vals_p = jnp.concatenate(
        [mat_xy_values, jnp.zeros((pad,), jnp.float32)]
    ).astype(jnp.bfloat16).reshape(NS, M, RB)

    out = _sc_scatter(table, packed_p, vals_p)
    return out.reshape(NX, NY, NZ)


# speedup vs baseline: 1.7283x; 1.7283x over previous
import jax
import jax.numpy as jnp
from jax import lax
from jax.experimental import pallas as pl
from jax.experimental.pallas import tpu as pltpu
from jax.experimental.pallas import tpu_sc as plsc

NX, NY, NZ = 128, 128, 256
N = NX * NY
NNZ = 268435

NC = 2
NS = 16
CW = 32
NCHUNK = NZ // CW
CHUNKS_PER_SC = NCHUNK // NC
BATCH = 128
ROWS_PER_TILE = N // NS
K = 3
PD = 2
R = 4
RB = R * BATCH
M = ((-(-NNZ // (NS * RB)) + K - 1) // K) * K
NNZ_PAD = NS * M * RB


def _mm_body(x_ref, w_ref, o_ref):
    o_ref[...] = jnp.dot(x_ref[...], w_ref[...],
                         preferred_element_type=jnp.float32)


def _matmul(squ, mat_z):
    BM = 1024
    return pl.pallas_call(
        _mm_body,
        grid=(N // BM,),
        in_specs=[
            pl.BlockSpec((BM, NZ), lambda i: (i, 0)),
            pl.BlockSpec((NZ, NZ), lambda i: (0, 0)),
        ],
        out_specs=pl.BlockSpec((BM, NZ), lambda i: (i, 0)),
        out_shape=jax.ShapeDtypeStruct((N, NZ), jnp.float32),
    )(squ, mat_z)


def _sc_body(table, packed_h, vals_h, out,
             idx_v, cols_v, vals_v,
             data0, data1, data2, acc_ref,
             gs0, gs1, gs2, ss0, ss1, ss2):
    c = lax.axis_index("c")
    s = lax.axis_index("s")
    data = (data0, data1, data2)
    gsem = (gs0, gs1, gs2)
    ssem = (ss0, ss1, ss2)

    pltpu.sync_copy(packed_h.at[s], idx_v)
    pltpu.sync_copy(vals_h.at[s], vals_v)

    maskv = jnp.full((16,), N - 1, jnp.int32)
    zvec = jnp.zeros((16,), jnp.float32)

    def _unpack_body(b, _):
        for j in range(RB // 16):
            w = idx_v[b, pl.ds(j * 16, 16)]
            cols_v[b, pl.ds(j * 16, 16)] = w & maskv
            idx_v[b, pl.ds(j * 16, 16)] = (
                lax.shift_right_logical(w, jnp.full((16,), 11, jnp.int32))
                & jnp.full((16,), ~7, jnp.int32))
        return 0

    lax.fori_loop(0, M, _unpack_body, 0)

    for i in range(CHUNKS_PER_SC):
        chunk = CHUNKS_PER_SC * c + i
        delta = chunk if i == 0 else 1
        deltav = lax.broadcast(jnp.int32(delta), (16,))

        def _shift_body(b, _):
            for j in range(RB // 16):
                idx_v[b, pl.ds(j * 16, 16)] = (
                    idx_v[b, pl.ds(j * 16, 16)] + deltav)
            return 0

        lax.fori_loop(0, M, _shift_body, 0)

        def _zero_body(r, _):
            for k in range(CW // 16):
                data0[r, pl.ds(k * 16, 16)] = zvec
            return 0

        lax.fori_loop(0, BATCH, _zero_body, 0)
        for z in range(ROWS_PER_TILE // BATCH):
            pltpu.sync_copy(
                data0.at[pl.ds(0, BATCH)],
                acc_ref.at[pl.ds(s * ROWS_PER_TILE + z * BATCH, BATCH)])
        plsc.subcore_barrier()

        for k in range(PD):
            pltpu.async_copy(table.at[idx_v.at[k]],
                             data[k], gsem[k])

        dnums = lax.GatherDimensionNumbers(
            offset_dims=(), collapsed_slice_dims=(0,), start_index_map=(0,))

        def _splat(vec, l):
            return lax.gather(
                vec, jnp.full((16, 1), l, jnp.int32), dnums, (1,),
                mode=lax.GatherScatterMode.PROMISE_IN_BOUNDS)

        def _scale(buf, m):
            def _rows(g, _):
                raws = vals_v[m, pl.ds(g * 32, 32)]
                vlo, vhi = plsc.unpack(
                    raws, format=plsc.PackFormat.INTERLEAVED)
                for j2 in range(16):
                    for vv, off in ((vlo, 0), (vhi, 1)):
                        vsp = _splat(vv, j2)
                        r = g * 32 + 2 * j2 + off
                        for k in range(CW // 16):
                            buf[r, pl.ds(k * 16, 16)] = (
                                buf[r, pl.ds(k * 16, 16)] * vsp)
                return 0
            lax.fori_loop(0, RB // 32, _rows, 0)

        def _ring_body(g, _):
            for k in range(K):
                m = g * K + k
                pltpu.make_async_copy(table.at[idx_v.at[m]],
                                      data[k], gsem[k]).wait()
                _scale(data[k], m)
                pltpu.async_copy(
                    data[k], acc_ref.at[cols_v.at[m]],
                    ssem[k], add=True)
                kp = (k + PD) % K
                mp = m + PD

                @pl.when(mp < M)
                def _():
                    @pl.when(mp - K >= 0)
                    def _():
                        pltpu.make_async_copy(
                            data[kp],
                            acc_ref.at[cols_v.at[mp - K]],
                            ssem[kp]).wait()
                    pltpu.async_copy(table.at[idx_v.at[mp]],
                                     data[kp], gsem[kp])
            return 0

        lax.fori_loop(0, M // K, _ring_body, 0)
        for k in range(K):
            m = M - K + k
            pltpu.make_async_copy(
                data[(m % K)], acc_ref.at[cols_v.at[m]],
                ssem[m % K]).wait()
        plsc.subcore_barrier()

        pltpu.sync_copy(
            acc_ref.at[pl.ds(s * ROWS_PER_TILE, ROWS_PER_TILE)],
            out.at[pl.ds(s * ROWS_PER_TILE, ROWS_PER_TILE),
                   pl.ds(chunk * CW, CW)])
        plsc.subcore_barrier()


def _sc_scatter(table, packed_r, vals_r):
    mesh = plsc.VectorSubcoreMesh(core_axis_name="c", subcore_axis_name="s")
    f = pl.kernel(
        _sc_body,
        mesh=mesh,
        compiler_params=pltpu.CompilerParams(use_tc_tiling_on_sc=False,
                                             needs_layout_passes=False),
        out_type=jax.ShapeDtypeStruct((N, NZ), jnp.float32),
        scratch_types=[
            pltpu.VMEM((M, RB), jnp.int32),
            pltpu.VMEM((M, RB), jnp.int32),
            pltpu.VMEM((M, RB), jnp.bfloat16),
            pltpu.VMEM((RB, CW), jnp.float32),
            pltpu.VMEM((RB, CW), jnp.float32),
            pltpu.VMEM((RB, CW), jnp.float32),
            pltpu.VMEM_SHARED((N, CW), jnp.float32),
            pltpu.SemaphoreType.DMA,
            pltpu.SemaphoreType.DMA,
            pltpu.SemaphoreType.DMA,
            pltpu.SemaphoreType.DMA,
            pltpu.SemaphoreType.DMA,
            pltpu.SemaphoreType.DMA,
        ],
    )
    return f(table, packed_r, vals_r)


def kernel(bproj, mat_xy_indices, mat_xy_values, mat_z):
    squ = bproj.reshape(N, NZ)
    result1 = _matmul(squ, mat_z)
    table = result1.reshape(N * NCHUNK, CW)

    rows = mat_xy_indices[:, 0]
    cols = mat_xy_indices[:, 1]
    packed = rows * N + cols
    pad = NNZ_PAD - NNZ
    pad_idx = (jnp.arange(pad, dtype=jnp.int32) * 37) % N
    packed_p = jnp.concatenate(
        [packed, pad_idx * N + pad_idx]).reshape(NS, M, RB)
    vals_p = jnp.concatenate(
        [mat_xy_values, jnp.zeros((pad,), jnp.float32)]
    ).astype(jnp.bfloat16).reshape(NS, M, RB)

    out = _sc_scatter(table, packed_p, vals_p)
    return out.reshape(NX, NY, NZ)


# speedup vs baseline: 1.7425x; 1.0082x over previous
import jax
import jax.numpy as jnp
from jax import lax
from jax.experimental import pallas as pl
from jax.experimental.pallas import tpu as pltpu
from jax.experimental.pallas import tpu_sc as plsc

NX, NY, NZ = 128, 128, 256
N = NX * NY
NNZ = 268435

NC = 2
NS = 16
CW = 32
NCHUNK = NZ // CW
CHUNKS_PER_SC = NCHUNK // NC
BATCH = 128
ROWS_PER_TILE = N // NS
K = 3
PD = 2
R = 2
RB = R * BATCH
M = ((-(-NNZ // (NS * RB)) + K - 1) // K) * K
NNZ_PAD = NS * M * RB


def _mm_body(x_ref, w_ref, o_ref):
    o_ref[...] = jnp.dot(x_ref[...], w_ref[...],
                         preferred_element_type=jnp.float32)


def _matmul(squ, mat_z):
    BM = 1024
    return pl.pallas_call(
        _mm_body,
        grid=(N // BM,),
        in_specs=[
            pl.BlockSpec((BM, NZ), lambda i: (i, 0)),
            pl.BlockSpec((NZ, NZ), lambda i: (0, 0)),
        ],
        out_specs=pl.BlockSpec((BM, NZ), lambda i: (i, 0)),
        out_shape=jax.ShapeDtypeStruct((N, NZ), jnp.float32),
    )(squ, mat_z)


def _sc_body(table, packed_h, vals_h, out,
             idx_v, cols_v, vals_v,
             data0, data1, data2, acc_ref,
             gs0, gs1, gs2, ss0, ss1, ss2):
    c = lax.axis_index("c")
    s = lax.axis_index("s")
    data = (data0, data1, data2)
    gsem = (gs0, gs1, gs2)
    ssem = (ss0, ss1, ss2)

    pltpu.sync_copy(packed_h.at[s], idx_v)
    pltpu.sync_copy(vals_h.at[s], vals_v)

    maskv = jnp.full((16,), N - 1, jnp.int32)
    zvec = jnp.zeros((16,), jnp.float32)

    def _unpack_body(b, _):
        for j in range(RB // 16):
            w = idx_v[b, pl.ds(j * 16, 16)]
            cols_v[b, pl.ds(j * 16, 16)] = w & maskv
            idx_v[b, pl.ds(j * 16, 16)] = (
                lax.shift_right_logical(w, jnp.full((16,), 11, jnp.int32))
                & jnp.full((16,), ~7, jnp.int32))
        return 0

    lax.fori_loop(0, M, _unpack_body, 0)

    for i in range(CHUNKS_PER_SC):
        chunk = CHUNKS_PER_SC * c + i
        delta = chunk if i == 0 else 1
        deltav = lax.broadcast(jnp.int32(delta), (16,))

        def _shift_body(b, _):
            for j in range(RB // 16):
                idx_v[b, pl.ds(j * 16, 16)] = (
                    idx_v[b, pl.ds(j * 16, 16)] + deltav)
            return 0

        lax.fori_loop(0, M, _shift_body, 0)

        def _zero_body(r, _):
            for k in range(CW // 16):
                data0[r, pl.ds(k * 16, 16)] = zvec
            return 0

        lax.fori_loop(0, BATCH, _zero_body, 0)
        for z in range(ROWS_PER_TILE // BATCH):
            pltpu.sync_copy(
                data0.at[pl.ds(0, BATCH)],
                acc_ref.at[pl.ds(s * ROWS_PER_TILE + z * BATCH, BATCH)])
        plsc.subcore_barrier()

        for k in range(PD):
            pltpu.async_copy(table.at[idx_v.at[k]],
                             data[k], gsem[k])

        dnums = lax.GatherDimensionNumbers(
            offset_dims=(), collapsed_slice_dims=(0,), start_index_map=(0,))

        def _splat(vec, l):
            return lax.gather(
                vec, jnp.full((16, 1), l, jnp.int32), dnums, (1,),
                mode=lax.GatherScatterMode.PROMISE_IN_BOUNDS)

        def _scale(buf, m):
            def _rows(j, _):
                vv = vals_v[m, pl.ds(j * 16, 16)]
                for l in range(16):
                    vsp = _splat(vv, l)
                    r = j * 16 + l
                    for k in range(CW // 16):
                        buf[r, pl.ds(k * 16, 16)] = (
                            buf[r, pl.ds(k * 16, 16)] * vsp)
                return 0
            lax.fori_loop(0, RB // 16, _rows, 0)

        def _ring_body(g, _):
            for k in range(K):
                m = g * K + k
                pltpu.make_async_copy(table.at[idx_v.at[m]],
                                      data[k], gsem[k]).wait()
                _scale(data[k], m)
                pltpu.async_copy(
                    data[k], acc_ref.at[cols_v.at[m]],
                    ssem[k], add=True)
                kp = (k + PD) % K
                mp = m + PD

                @pl.when(mp < M)
                def _():
                    @pl.when(mp - K >= 0)
                    def _():
                        pltpu.make_async_copy(
                            data[kp],
                            acc_ref.at[cols_v.at[mp - K]],
                            ssem[kp]).wait()
                    pltpu.async_copy(table.at[idx_v.at[mp]],
                                     data[kp], gsem[kp])
            return 0

        lax.fori_loop(0, M // K, _ring_body, 0)
        for k in range(K):
            m = M - K + k
            pltpu.make_async_copy(
                data[(m % K)], acc_ref.at[cols_v.at[m]],
                ssem[m % K]).wait()
        plsc.subcore_barrier()

        pltpu.sync_copy(
            acc_ref.at[pl.ds(s * ROWS_PER_TILE, ROWS_PER_TILE)],
            out.at[pl.ds(s * ROWS_PER_TILE, ROWS_PER_TILE),
                   pl.ds(chunk * CW, CW)])
        plsc.subcore_barrier()


def _sc_scatter(table, packed_r, vals_r):
    mesh = plsc.VectorSubcoreMesh(core_axis_name="c", subcore_axis_name="s")
    f = pl.kernel(
        _sc_body,
        mesh=mesh,
        compiler_params=pltpu.CompilerParams(use_tc_tiling_on_sc=False,
                                             needs_layout_passes=False),
        out_type=jax.ShapeDtypeStruct((N, NZ), jnp.float32),
        scratch_types=[
            pltpu.VMEM((M, RB), jnp.int32),
            pltpu.VMEM((M, RB), jnp.int32),
            pltpu.VMEM((M, RB), jnp.float32),
            pltpu.VMEM((RB, CW), jnp.float32),
            pltpu.VMEM((RB, CW), jnp.float32),
            pltpu.VMEM((RB, CW), jnp.float32),
            pltpu.VMEM_SHARED((N, CW), jnp.float32),
            pltpu.SemaphoreType.DMA,
            pltpu.SemaphoreType.DMA,
            pltpu.SemaphoreType.DMA,
            pltpu.SemaphoreType.DMA,
            pltpu.SemaphoreType.DMA,
            pltpu.SemaphoreType.DMA,
        ],
    )
    return f(table, packed_r, vals_r)


def kernel(bproj, mat_xy_indices, mat_xy_values, mat_z):
    squ = bproj.reshape(N, NZ)
    result1 = _matmul(squ, mat_z)
    table = result1.reshape(N * NCHUNK, CW)

    rows = mat_xy_indices[:, 0]
    cols = mat_xy_indices[:, 1]
    packed = rows * N + cols
    pad = NNZ_PAD - NNZ
    pad_idx = (jnp.arange(pad, dtype=jnp.int32) * 37) % N
    packed_p = jnp.concatenate(
        [packed, pad_idx * N + pad_idx]).reshape(NS, M, RB)
    vals_p = jnp.concatenate(
        [mat_xy_values, jnp.zeros((pad,), jnp.float32)]).reshape(NS, M, RB)

    out = _sc_scatter(table, packed_p, vals_p)
    return out.reshape(NX, NY, NZ)
